# baseline (speedup 1.0000x reference)
import functools

import jax
import jax.numpy as jnp
from jax import lax
from jax.experimental import pallas as pl
from jax.experimental.pallas import tpu as pltpu
from jax.experimental.pallas import tpu_sc as plsc

N = 10000
E = 320000
D = 128
F = 9
V = 119
G = 64

NC = 2
NS = 16
NW = NC * NS

NPAD = 10240
ROWS_PER_TILE = NPAD // NS
NODES_PER_W = NPAD // NW

CHUNK = 128
E_PAD = 327680
NCHUNK_T = E_PAD // NS // CHUNK
NHALF = NPAD // NC
NRING = 4

BLK_M = 1280
NBLK_M = NPAD // BLK_M

BLK = 2000
NBLK = N // BLK

_mesh = plsc.VectorSubcoreMesh(core_axis_name="c", subcore_axis_name="s")


def _zero_vmem(ref, rows):
    def body(i, carry):
        for j in range(D // 16):
            ref[i, pl.ds(j * 16, 16)] = jnp.zeros((16,), jnp.float32)
        return carry
    lax.fori_loop(0, rows, body, 0)


@functools.partial(
    pl.kernel,
    out_type=jax.ShapeDtypeStruct((NPAD, D), jnp.float32),
    mesh=_mesh,
    scratch_types=[
        pltpu.VMEM((40, 80), jnp.int32),
        pltpu.VMEM((NODES_PER_W, D), jnp.float32),
        pltpu.VMEM((NODES_PER_W, D), jnp.float32),
        pltpu.SemaphoreType.DMA,
    ],
)
def _atom_encode(idx_hbm, tab_hbm, out_hbm, idx_v, acc_v, gbuf_v, sem):
    c = lax.axis_index("c")
    s = lax.axis_index("s")
    w = s * NC + c
    pltpu.sync_copy(idx_hbm.at[pl.ds(w * 40, 40)], idx_v)
    _zero_vmem(acc_v, NODES_PER_W)

    def feat_body(f, carry):
        for cc in range(4):
            pltpu.make_async_copy(
                tab_hbm.at[idx_v.at[f * 4 + cc]],
                gbuf_v.at[pl.ds(cc * 80, 80)],
                sem,
            ).start()
        for cc in range(4):
            pltpu.make_async_copy(
                tab_hbm.at[idx_v.at[f * 4 + cc]],
                gbuf_v.at[pl.ds(cc * 80, 80)],
                sem,
            ).wait()

        def addrow(i, carry2):
            for j in range(D // 16):
                sl = pl.ds(j * 16, 16)
                acc_v[i, sl] = acc_v[i, sl] + gbuf_v[i, sl]
            return carry2
        lax.fori_loop(0, NODES_PER_W, addrow, 0)
        return carry

    lax.fori_loop(0, F, feat_body, 0)

    def fixrow(i, carry2):
        keep = jnp.where(w * NODES_PER_W + i < N, 1.0, 0.0)
        for j in range(D // 16):
            sl = pl.ds(j * 16, 16)
            acc_v[i, sl] = acc_v[i, sl] * keep
        return carry2
    lax.fori_loop(0, NODES_PER_W, fixrow, 0)

    pltpu.sync_copy(acc_v, out_hbm.at[pl.ds(w * NODES_PER_W, NODES_PER_W)])


@functools.partial(
    pl.kernel,
    out_type=jax.ShapeDtypeStruct((NPAD, D), jnp.float32),
    mesh=_mesh,
    scratch_types=[
        pltpu.VMEM((NCHUNK_T, CHUNK), jnp.int32),
        pltpu.VMEM((NRING, CHUNK), jnp.int32),
        pltpu.VMEM((NRING, CHUNK), jnp.int32),
        pltpu.VMEM((CHUNK, D), jnp.float32),
        pltpu.VMEM((CHUNK, D), jnp.float32),
        pltpu.VMEM((CHUNK, D), jnp.float32),
        pltpu.VMEM((CHUNK, D), jnp.float32),
        pltpu.VMEM_SHARED((NHALF, D), jnp.float32),
        pltpu.SemaphoreType.DMA,
        pltpu.SemaphoreType.DMA,
        pltpu.SemaphoreType.DMA,
        pltpu.SemaphoreType.DMA,
    ],
)
def _edge_agg(h_hbm, se_hbm, out_hbm, pk_v, si_v, di_v,
              b0, b1, b2, b3, agg_sh, gsx, gsy, ssx, ssy):
    c = lax.axis_index("c")
    s = lax.axis_index("s")
    pltpu.sync_copy(se_hbm.at[pl.ds(s * NCHUNK_T, NCHUNK_T)], pk_v)

    base = c * NHALF
    lane = lax.iota(jnp.int32, 16)

    def unpack(j, k):
        for jj in range(CHUNK // 16):
            sl = pl.ds(jj * 16, 16)
            p = pk_v[j, sl]
            rel = (p & 16383) - base
            ok = (rel >= 0) & (rel < NHALF)
            spread = ((j * CHUNK + jj * 16) % 4096) + lane
            si_v[k, sl] = jnp.where(ok, p >> 14, N + (spread % (NPAD - N)))
            di_v[k, sl] = jnp.where(ok, rel, spread)

    for b in (b0, b2, b3):
        _zero_vmem(b, CHUNK)
    rpt = NHALF // NS
    for k in range(2):
        pltpu.sync_copy(b0, agg_sh.at[pl.ds(s * rpt + k * CHUNK, CHUNK)])
    pltpu.sync_copy(b0.at[pl.ds(0, rpt - 2 * CHUNK)],
                    agg_sh.at[pl.ds(s * rpt + 2 * CHUNK, rpt - 2 * CHUNK)])
    for k in range(NRING):
        unpack(k, k)
    plsc.subcore_barrier()

    def g_start(k, b, sem):
        pltpu.async_copy(h_hbm.at[si_v.at[k]], b, sem)

    def g_wait(k, b, sem):
        pltpu.make_async_copy(h_hbm.at[si_v.at[k]], b, sem).wait()

    def s_start(k, b, sem):
        pltpu.async_copy(b, agg_sh.at[di_v.at[k]], sem, add=True)

    def s_wait(k, b, sem):
        pltpu.make_async_copy(b, agg_sh.at[di_v.at[k]], sem).wait()

    g_start(0, b0, gsx)
    g_start(1, b1, gsx)
    s_start(2, b2, ssy)
    s_start(3, b3, ssy)

    def body(jj, carry):
        j0 = 4 * jj
        g_wait(0, b0, gsx)
        g_wait(1, b1, gsx)
        s_wait(2, b2, ssy)
        s_wait(3, b3, ssy)
        unpack(j0 + 2, 2)
        unpack(j0 + 3, 3)
        g_start(2, b2, gsy)
        g_start(3, b3, gsy)
        s_start(0, b0, ssx)
        s_start(1, b1, ssx)
        g_wait(2, b2, gsy)
        g_wait(3, b3, gsy)
        s_wait(0, b0, ssx)
        s_wait(1, b1, ssx)
        unpack(jnp.minimum(j0 + 4, NCHUNK_T - 1), 0)
        unpack(jnp.minimum(j0 + 5, NCHUNK_T - 1), 1)
        g_start(0, b0, gsx)
        g_start(1, b1, gsx)
        s_start(2, b2, ssy)
        s_start(3, b3, ssy)
        return carry

    lax.fori_loop(0, NCHUNK_T // 4, body, 0)
    g_wait(0, b0, gsx)
    g_wait(1, b1, gsx)
    s_wait(2, b2, ssy)
    s_wait(3, b3, ssy)

    plsc.subcore_barrier()
    pltpu.sync_copy(
        agg_sh.at[pl.ds(s * rpt, rpt)],
        out_hbm.at[pl.ds(c * NHALF + s * rpt, rpt)])


def _mlp_body(h_ref, agg_ref, w1_ref, b1_ref, w2_ref, b2_ref, out_ref):
    i = pl.program_id(0)
    z = h_ref[...] + agg_ref[...]
    a = jnp.maximum(
        jnp.dot(z, w1_ref[...], preferred_element_type=jnp.float32)
        + b1_ref[...], 0.0)
    y = jnp.maximum(
        jnp.dot(a, w2_ref[...], preferred_element_type=jnp.float32)
        + b2_ref[...], 0.0)
    rows = lax.broadcasted_iota(jnp.int32, (BLK_M, D), 0) + i * BLK_M
    out_ref[...] = jnp.where(rows < N, y, 0.0)


def _mlp(h, agg, W1, b1, W2, b2):
    return pl.pallas_call(
        _mlp_body,
        grid=(NBLK_M,),
        in_specs=[
            pl.BlockSpec((BLK_M, D), lambda i: (i, 0)),
            pl.BlockSpec((BLK_M, D), lambda i: (i, 0)),
            pl.BlockSpec((D, D), lambda i: (0, 0)),
            pl.BlockSpec((1, D), lambda i: (0, 0)),
            pl.BlockSpec((D, D), lambda i: (0, 0)),
            pl.BlockSpec((1, D), lambda i: (0, 0)),
        ],
        out_specs=pl.BlockSpec((BLK_M, D), lambda i: (i, 0)),
        out_shape=jax.ShapeDtypeStruct((NPAD, D), jnp.float32),
    )(h, agg, W1, b1.reshape(1, D), W2, b2.reshape(1, D))


def _readout_body(h_ref, bidx_ref, wl_ref, bl_ref, out_ref, sums_ref, cnt_ref):
    i = pl.program_id(0)

    @pl.when(i == 0)
    def _init():
        sums_ref[...] = jnp.zeros_like(sums_ref)
        cnt_ref[...] = jnp.zeros_like(cnt_ref)

    y = h_ref[...]
    bidx = bidx_ref[0]
    gid = lax.broadcasted_iota(jnp.int32, (G, BLK), 0)
    m = (gid == bidx).astype(jnp.float32)
    sums_ref[...] += jnp.dot(m, y, preferred_element_type=jnp.float32,
                             precision=lax.Precision.HIGHEST)
    cnt_ref[...] += jnp.broadcast_to(
        jnp.sum(m, axis=1, keepdims=True), (G, D))

    @pl.when(i == NBLK - 1)
    def _fin():
        mean = sums_ref[...] / jnp.maximum(cnt_ref[...], 1.0)
        proj = jnp.dot(mean, wl_ref[...], preferred_element_type=jnp.float32)
        out_ref[...] = jax.nn.sigmoid(proj + bl_ref[...])


def _readout(h, bidx3d, wl_pad, bl11):
    return pl.pallas_call(
        _readout_body,
        grid=(NBLK,),
        in_specs=[
            pl.BlockSpec((BLK, D), lambda i: (i, 0)),
            pl.BlockSpec((1, 1, BLK), lambda i: (i, 0, 0)),
            pl.BlockSpec((D, D), lambda i: (0, 0)),
            pl.BlockSpec((1, 1), lambda i: (0, 0)),
        ],
        out_specs=pl.BlockSpec((G, D), lambda i: (0, 0)),
        out_shape=jax.ShapeDtypeStruct((G, D), jnp.float32),
        scratch_shapes=[
            pltpu.VMEM((G, D), jnp.float32),
            pltpu.VMEM((G, D), jnp.float32),
        ],
    )(h, bidx3d, wl_pad, bl11)


def kernel(x, edge_index, batch_idx, atom_emb,
           W1_1, b1_1, W2_1, b2_1, W1_2, b1_2, W2_2, b2_2,
           W1_3, b1_3, W2_3, b2_3, Wl, bl):
    offs = (jnp.arange(F, dtype=jnp.int32) * V)[None, :]
    xf = (x + offs).T
    pad = (jnp.arange(NPAD - N, dtype=jnp.int32) % (F * V))[None, :]
    xf = jnp.concatenate(
        [xf, jnp.broadcast_to(pad, (F, NPAD - N))], axis=1)
    idx_a = xf.reshape(F, NW, 4, 80).transpose(1, 0, 2, 3).reshape(NW, F * 4, 80)
    idx_fill = jnp.broadcast_to(
        (jnp.arange(80, dtype=jnp.int32) % (F * V))[None, None, :], (NW, 4, 80))
    idx_a = jnp.concatenate([idx_a, idx_fill], axis=1).reshape(NW * 40, 80)
    tab = atom_emb.reshape(F * V, D)
    h0 = _atom_encode(idx_a, tab)

    npe = E_PAD - E
    pad_src = jnp.arange(npe, dtype=jnp.int32) % N
    pad_dst = jnp.full((npe,), NPAD, jnp.int32)
    se_i = jnp.concatenate(
        [edge_index[0] * 16384 + edge_index[1], pad_src * 16384 + pad_dst]
    ).reshape(NS * NCHUNK_T, CHUNK)

    bidx3d = batch_idx.reshape(NBLK, 1, BLK)
    wl_pad = jnp.pad(Wl, ((0, 0), (0, D - 1)))
    bl11 = bl.reshape(1, 1)

    ws = (jnp.stack([W1_1, W1_2, W1_3]), jnp.stack([b1_1, b1_2, b1_3]),
          jnp.stack([W2_1, W2_2, W2_3]), jnp.stack([b2_1, b2_2, b2_3]))

    def layer(h, wts):
        w1, bb1, w2, bb2 = wts
        agg = _edge_agg(h, se_i)
        return _mlp(h, agg, w1, bb1, w2, bb2), None

    h, _ = lax.scan(layer, h0, ws)
    out_full = _readout(h, bidx3d, wl_pad, bl11)
    return out_full[:, :1]


# speedup vs baseline: 1.0120x; 1.0120x over previous
import functools

import jax
import jax.numpy as jnp
from jax import lax
from jax.experimental import pallas as pl
from jax.experimental.pallas import tpu as pltpu
from jax.experimental.pallas import tpu_sc as plsc

N = 10000
E = 320000
D = 128
F = 9
V = 119
G = 64

NC = 2
NS = 16
NW = NC * NS

NPAD = 10240
ROWS_PER_TILE = NPAD // NS
NODES_PER_W = NPAD // NW

CHUNK = 128
E_PAD = 327680
NCHUNK_T = E_PAD // NS // CHUNK
NHALF = NPAD // NC
NRING = 4

BLK_M = 1280
NBLK_M = NPAD // BLK_M

BLK = 2000
NBLK = N // BLK

_mesh = plsc.VectorSubcoreMesh(core_axis_name="c", subcore_axis_name="s")


def _zero_vmem(ref, rows):
    def body(i, carry):
        for j in range(D // 16):
            ref[i, pl.ds(j * 16, 16)] = jnp.zeros((16,), jnp.float32)
        return carry
    lax.fori_loop(0, rows, body, 0)


@functools.partial(
    pl.kernel,
    out_type=jax.ShapeDtypeStruct((NPAD, D), jnp.float32),
    mesh=_mesh,
    scratch_types=[
        pltpu.VMEM((40, 80), jnp.int32),
        pltpu.VMEM((NODES_PER_W, D), jnp.float32),
        pltpu.VMEM((NODES_PER_W, D), jnp.float32),
        pltpu.VMEM((NODES_PER_W, D), jnp.float32),
        pltpu.SemaphoreType.DMA,
        pltpu.SemaphoreType.DMA,
    ],
)
def _atom_encode(idx_hbm, tab_hbm, out_hbm, idx_v, acc_v, g0, g1, sm0, sm1):
    c = lax.axis_index("c")
    s = lax.axis_index("s")
    w = s * NC + c
    pltpu.sync_copy(idx_hbm.at[pl.ds(w * 40, 40)], idx_v)
    _zero_vmem(acc_v, NODES_PER_W)

    def fire(f, gb, sem):
        for cc in range(4):
            pltpu.make_async_copy(
                tab_hbm.at[idx_v.at[f * 4 + cc]],
                gb.at[pl.ds(cc * 80, 80)], sem).start()

    def drain(f, gb, sem):
        for cc in range(4):
            pltpu.make_async_copy(
                tab_hbm.at[idx_v.at[f * 4 + cc]],
                gb.at[pl.ds(cc * 80, 80)], sem).wait()

    def accadd(gb):
        def addrow(i, carry2):
            for j in range(D // 16):
                sl = pl.ds(j * 16, 16)
                acc_v[i, sl] = acc_v[i, sl] + gb[i, sl]
            return carry2
        lax.fori_loop(0, NODES_PER_W, addrow, 0)

    fire(0, g0, sm0)

    def feat_body(ff, carry):
        f0 = 2 * ff
        drain(f0, g0, sm0)
        fire(f0 + 1, g1, sm1)
        accadd(g0)
        drain(f0 + 1, g1, sm1)
        fire(jnp.minimum(f0 + 2, F - 1), g0, sm0)
        accadd(g1)
        return carry

    lax.fori_loop(0, F // 2, feat_body, 0)
    drain(F - 1, g0, sm0)
    accadd(g0)

    def fixrow(i, carry2):
        keep = jnp.where(w * NODES_PER_W + i < N, 1.0, 0.0)
        for j in range(D // 16):
            sl = pl.ds(j * 16, 16)
            acc_v[i, sl] = acc_v[i, sl] * keep
        return carry2
    lax.fori_loop(0, NODES_PER_W, fixrow, 0)

    pltpu.sync_copy(acc_v, out_hbm.at[pl.ds(w * NODES_PER_W, NODES_PER_W)])


@functools.partial(
    pl.kernel,
    out_type=jax.ShapeDtypeStruct((NPAD, D), jnp.float32),
    mesh=_mesh,
    scratch_types=[
        pltpu.VMEM((NCHUNK_T, CHUNK), jnp.int32),
        pltpu.VMEM((NCHUNK_T, CHUNK), jnp.int32),
        pltpu.VMEM((CHUNK, D), jnp.float32),
        pltpu.VMEM((CHUNK, D), jnp.float32),
        pltpu.VMEM_SHARED((NHALF, D), jnp.float32),
        pltpu.SemaphoreType.DMA,
        pltpu.SemaphoreType.DMA,
    ],
)
def _edge_agg(h_hbm, se_hbm, out_hbm, src_v, dst_v,
              buf0, buf1, agg_sh, sem0, sem1):
    c = lax.axis_index("c")
    s = lax.axis_index("s")
    pltpu.sync_copy(se_hbm.at[pl.ds(s * NCHUNK_T, NCHUNK_T)], dst_v)

    base = c * NHALF
    lane = lax.iota(jnp.int32, 16)

    def unpack(i, carry):
        for j in range(CHUNK // 16):
            sl = pl.ds(j * 16, 16)
            p = dst_v[i, sl]
            rel = (p & 16383) - base
            ok = (rel >= 0) & (rel < NHALF)
            spread = ((i * CHUNK + j * 16) % 4096) + lane
            src_v[i, sl] = jnp.where(ok, p >> 14, N + (spread % (NPAD - N)))
            dst_v[i, sl] = jnp.where(ok, rel, spread)
        return carry

    lax.fori_loop(0, NCHUNK_T, unpack, 0)

    _zero_vmem(buf0, CHUNK)
    rpt = NHALF // NS
    for k in range(2):
        pltpu.sync_copy(buf0, agg_sh.at[pl.ds(s * rpt + k * CHUNK, CHUNK)])
    pltpu.sync_copy(buf0.at[pl.ds(0, rpt - 2 * CHUNK)],
                    agg_sh.at[pl.ds(s * rpt + 2 * CHUNK, rpt - 2 * CHUNK)])
    plsc.subcore_barrier()

    def g_start(j, buf, sem):
        pltpu.make_async_copy(h_hbm.at[src_v.at[j]], buf, sem).start()

    def g_wait(j, buf, sem):
        pltpu.make_async_copy(h_hbm.at[src_v.at[j]], buf, sem).wait()

    def scat(j, buf):
        pltpu.sync_copy(buf, agg_sh.at[dst_v.at[j]], add=True)

    g_start(0, buf0, sem0)

    def body(jj, carry):
        j0 = 2 * jj
        g_start(j0 + 1, buf1, sem1)
        g_wait(j0, buf0, sem0)
        scat(j0, buf0)
        g_start(j0 + 2, buf0, sem0)
        g_wait(j0 + 1, buf1, sem1)
        scat(j0 + 1, buf1)
        return carry

    lax.fori_loop(0, NCHUNK_T // 2 - 1, body, 0)
    g_start(NCHUNK_T - 1, buf1, sem1)
    g_wait(NCHUNK_T - 2, buf0, sem0)
    scat(NCHUNK_T - 2, buf0)
    g_wait(NCHUNK_T - 1, buf1, sem1)
    scat(NCHUNK_T - 1, buf1)

    plsc.subcore_barrier()
    pltpu.sync_copy(
        agg_sh.at[pl.ds(s * rpt, rpt)],
        out_hbm.at[pl.ds(c * NHALF + s * rpt, rpt)])


def _mlp_body(h_ref, agg_ref, w1_ref, b1_ref, w2_ref, b2_ref, out_ref):
    i = pl.program_id(0)
    z = h_ref[...] + agg_ref[...]
    a = jnp.maximum(
        jnp.dot(z, w1_ref[...], preferred_element_type=jnp.float32)
        + b1_ref[...], 0.0)
    y = jnp.maximum(
        jnp.dot(a, w2_ref[...], preferred_element_type=jnp.float32)
        + b2_ref[...], 0.0)
    rows = lax.broadcasted_iota(jnp.int32, (BLK_M, D), 0) + i * BLK_M
    out_ref[...] = jnp.where(rows < N, y, 0.0)


def _mlp(h, agg, W1, b1, W2, b2):
    return pl.pallas_call(
        _mlp_body,
        grid=(NBLK_M,),
        in_specs=[
            pl.BlockSpec((BLK_M, D), lambda i: (i, 0)),
            pl.BlockSpec((BLK_M, D), lambda i: (i, 0)),
            pl.BlockSpec((D, D), lambda i: (0, 0)),
            pl.BlockSpec((1, D), lambda i: (0, 0)),
            pl.BlockSpec((D, D), lambda i: (0, 0)),
            pl.BlockSpec((1, D), lambda i: (0, 0)),
        ],
        out_specs=pl.BlockSpec((BLK_M, D), lambda i: (i, 0)),
        out_shape=jax.ShapeDtypeStruct((NPAD, D), jnp.float32),
    )(h, agg, W1, b1.reshape(1, D), W2, b2.reshape(1, D))


def _readout_body(h_ref, bidx_ref, wl_ref, bl_ref, out_ref, sums_ref, cnt_ref):
    i = pl.program_id(0)

    @pl.when(i == 0)
    def _init():
        sums_ref[...] = jnp.zeros_like(sums_ref)
        cnt_ref[...] = jnp.zeros_like(cnt_ref)

    y = h_ref[...]
    bidx = bidx_ref[0]
    gid = lax.broadcasted_iota(jnp.int32, (G, BLK), 0)
    m = (gid == bidx).astype(jnp.float32)
    sums_ref[...] += jnp.dot(m, y, preferred_element_type=jnp.float32,
                             precision=lax.Precision.HIGHEST)
    cnt_ref[...] += jnp.broadcast_to(
        jnp.sum(m, axis=1, keepdims=True), (G, D))

    @pl.when(i == NBLK - 1)
    def _fin():
        mean = sums_ref[...] / jnp.maximum(cnt_ref[...], 1.0)
        proj = jnp.dot(mean, wl_ref[...], preferred_element_type=jnp.float32)
        out_ref[...] = jax.nn.sigmoid(proj + bl_ref[...])


def _readout(h, bidx3d, wl_pad, bl11):
    return pl.pallas_call(
        _readout_body,
        grid=(NBLK,),
        in_specs=[
            pl.BlockSpec((BLK, D), lambda i: (i, 0)),
            pl.BlockSpec((1, 1, BLK), lambda i: (i, 0, 0)),
            pl.BlockSpec((D, D), lambda i: (0, 0)),
            pl.BlockSpec((1, 1), lambda i: (0, 0)),
        ],
        out_specs=pl.BlockSpec((G, D), lambda i: (0, 0)),
        out_shape=jax.ShapeDtypeStruct((G, D), jnp.float32),
        scratch_shapes=[
            pltpu.VMEM((G, D), jnp.float32),
            pltpu.VMEM((G, D), jnp.float32),
        ],
    )(h, bidx3d, wl_pad, bl11)


def kernel(x, edge_index, batch_idx, atom_emb,
           W1_1, b1_1, W2_1, b2_1, W1_2, b1_2, W2_2, b2_2,
           W1_3, b1_3, W2_3, b2_3, Wl, bl):
    offs = (jnp.arange(F, dtype=jnp.int32) * V)[None, :]
    xf = (x + offs).T
    pad = (jnp.arange(NPAD - N, dtype=jnp.int32) % (F * V))[None, :]
    xf = jnp.concatenate(
        [xf, jnp.broadcast_to(pad, (F, NPAD - N))], axis=1)
    idx_a = xf.reshape(F, NW, 4, 80).transpose(1, 0, 2, 3).reshape(NW, F * 4, 80)
    idx_fill = jnp.broadcast_to(
        (jnp.arange(80, dtype=jnp.int32) % (F * V))[None, None, :], (NW, 4, 80))
    idx_a = jnp.concatenate([idx_a, idx_fill], axis=1).reshape(NW * 40, 80)
    tab = atom_emb.reshape(F * V, D)
    h0 = _atom_encode(idx_a, tab)

    npe = E_PAD - E
    pad_src = jnp.arange(npe, dtype=jnp.int32) % N
    pad_dst = jnp.full((npe,), NPAD, jnp.int32)
    se_i = jnp.concatenate(
        [edge_index[0] * 16384 + edge_index[1], pad_src * 16384 + pad_dst]
    ).reshape(NS * NCHUNK_T, CHUNK)

    bidx3d = batch_idx.reshape(NBLK, 1, BLK)
    wl_pad = jnp.pad(Wl, ((0, 0), (0, D - 1)))
    bl11 = bl.reshape(1, 1)

    ws = (jnp.stack([W1_1, W1_2, W1_3]), jnp.stack([b1_1, b1_2, b1_3]),
          jnp.stack([W2_1, W2_2, W2_3]), jnp.stack([b2_1, b2_2, b2_3]))

    def layer(h, wts):
        w1, bb1, w2, bb2 = wts
        agg = _edge_agg(h, se_i)
        return _mlp(h, agg, w1, bb1, w2, bb2), None

    h, _ = lax.scan(layer, h0, ws)
    out_full = _readout(h, bidx3d, wl_pad, bl11)
    return out_full[:, :1]


# speedup vs baseline: 1.1056x; 1.0924x over previous
import functools

import jax
import jax.numpy as jnp
from jax import lax
from jax.experimental import pallas as pl
from jax.experimental.pallas import tpu as pltpu
from jax.experimental.pallas import tpu_sc as plsc

N = 10000
E = 320000
D = 128
F = 9
V = 119
G = 64

NC = 2
NS = 16
NW = NC * NS

NPAD = 10240
ROWS_PER_TILE = NPAD // NS
NODES_PER_W = NPAD // NW

CHUNK = 128
E_PAD = 327680
NCHUNK_T = E_PAD // NS // CHUNK
NHALF = NPAD // NC
NRING = 4

BLK_M = 1280
NBLK_M = NPAD // BLK_M

BLK = 2000
NBLK = N // BLK

_mesh = plsc.VectorSubcoreMesh(core_axis_name="c", subcore_axis_name="s")


def _zero_vmem(ref, rows):
    def body(i, carry):
        for j in range(D // 16):
            ref[i, pl.ds(j * 16, 16)] = jnp.zeros((16,), jnp.float32)
        return carry
    lax.fori_loop(0, rows, body, 0)


@functools.partial(
    pl.kernel,
    out_type=jax.ShapeDtypeStruct((NPAD, D), jnp.float32),
    mesh=_mesh,
    scratch_types=[
        pltpu.VMEM((40, 80), jnp.int32),
        pltpu.VMEM((NODES_PER_W, D), jnp.float32),
        pltpu.VMEM((NODES_PER_W, D), jnp.float32),
        pltpu.VMEM((NODES_PER_W, D), jnp.float32),
        pltpu.SemaphoreType.DMA,
        pltpu.SemaphoreType.DMA,
    ],
)
def _atom_encode(idx_hbm, tab_hbm, out_hbm, idx_v, acc_v, g0, g1, sm0, sm1):
    c = lax.axis_index("c")
    s = lax.axis_index("s")
    w = s * NC + c
    pltpu.sync_copy(idx_hbm.at[pl.ds(w * 40, 40)], idx_v)
    _zero_vmem(acc_v, NODES_PER_W)

    def fire(f, gb, sem):
        for cc in range(4):
            pltpu.make_async_copy(
                tab_hbm.at[idx_v.at[f * 4 + cc]],
                gb.at[pl.ds(cc * 80, 80)], sem).start()

    def drain(f, gb, sem):
        for cc in range(4):
            pltpu.make_async_copy(
                tab_hbm.at[idx_v.at[f * 4 + cc]],
                gb.at[pl.ds(cc * 80, 80)], sem).wait()

    def accadd(gb):
        def addrow(i, carry2):
            for j in range(D // 16):
                sl = pl.ds(j * 16, 16)
                acc_v[i, sl] = acc_v[i, sl] + gb[i, sl]
            return carry2
        lax.fori_loop(0, NODES_PER_W, addrow, 0)

    fire(0, g0, sm0)

    def feat_body(ff, carry):
        f0 = 2 * ff
        drain(f0, g0, sm0)
        fire(f0 + 1, g1, sm1)
        accadd(g0)
        drain(f0 + 1, g1, sm1)
        fire(jnp.minimum(f0 + 2, F - 1), g0, sm0)
        accadd(g1)
        return carry

    lax.fori_loop(0, F // 2, feat_body, 0)
    drain(F - 1, g0, sm0)
    accadd(g0)

    def fixrow(i, carry2):
        keep = jnp.where(w * NODES_PER_W + i < N, 1.0, 0.0)
        for j in range(D // 16):
            sl = pl.ds(j * 16, 16)
            acc_v[i, sl] = acc_v[i, sl] * keep
        return carry2
    lax.fori_loop(0, NODES_PER_W, fixrow, 0)

    pltpu.sync_copy(acc_v, out_hbm.at[pl.ds(w * NODES_PER_W, NODES_PER_W)])


@functools.partial(
    pl.kernel,
    out_type=jax.ShapeDtypeStruct((NPAD, D), jnp.float32),
    mesh=_mesh,
    scratch_types=[
        pltpu.VMEM((NCHUNK_T, CHUNK), jnp.int32),
        pltpu.VMEM((NCHUNK_T, CHUNK), jnp.int32),
        pltpu.VMEM((CHUNK, D), jnp.float32),
        pltpu.VMEM((CHUNK, D), jnp.float32),
        pltpu.VMEM((CHUNK, D), jnp.float32),
        pltpu.VMEM_SHARED((NHALF, D), jnp.float32),
        pltpu.SemaphoreType.DMA,
        pltpu.SemaphoreType.DMA,
    ],
)
def _edge_agg(h_hbm, src_hbm, dst_hbm, out_hbm, src_v, dst_v,
              buf0, buf1, zbuf, agg_sh, sem0, sem1):
    c = lax.axis_index("c")
    s = lax.axis_index("s")
    pltpu.sync_copy(src_hbm.at[pl.ds(s * NCHUNK_T, NCHUNK_T)], src_v)
    pltpu.sync_copy(dst_hbm.at[pl.ds(s * NCHUNK_T, NCHUNK_T)], dst_v)

    base = c * NHALF
    lane = lax.iota(jnp.int32, 16)

    def remap(i, carry):
        for j in range(CHUNK // 16):
            sl = pl.ds(j * 16, 16)
            rel = dst_v[i, sl] - base
            ok = (rel >= 0) & (rel < NHALF)
            spread = ((i * CHUNK + j * 16) % 4096) + lane
            dst_v[i, sl] = jnp.where(ok, rel, spread)
            src_v[i, sl] = jnp.where(ok, src_v[i, sl], N + (spread % (NPAD - N)))
        return carry

    lax.fori_loop(0, NCHUNK_T, remap, 0)

    _zero_vmem(zbuf, CHUNK)
    rpt = NHALF // NS
    for k in range(2):
        pltpu.sync_copy(zbuf, agg_sh.at[pl.ds(s * rpt + k * CHUNK, CHUNK)])
    pltpu.sync_copy(zbuf.at[pl.ds(0, rpt - 2 * CHUNK)],
                    agg_sh.at[pl.ds(s * rpt + 2 * CHUNK, rpt - 2 * CHUNK)])
    plsc.subcore_barrier()

    def g_start(j, buf, sem):
        pltpu.make_async_copy(h_hbm.at[src_v.at[j]], buf, sem).start()

    def g_wait(j, buf, sem):
        pltpu.make_async_copy(h_hbm.at[src_v.at[j]], buf, sem).wait()

    def scat(j, buf):
        pltpu.sync_copy(buf, agg_sh.at[dst_v.at[j]], add=True)

    g_start(0, buf0, sem0)

    def body(jj, carry):
        j0 = 2 * jj
        g_start(j0 + 1, buf1, sem1)
        g_wait(j0, buf0, sem0)
        scat(j0, buf0)
        g_start(j0 + 2, buf0, sem0)
        g_wait(j0 + 1, buf1, sem1)
        scat(j0 + 1, buf1)
        return carry

    lax.fori_loop(0, NCHUNK_T // 2 - 1, body, 0)
    g_start(NCHUNK_T - 1, buf1, sem1)
    g_wait(NCHUNK_T - 2, buf0, sem0)
    scat(NCHUNK_T - 2, buf0)
    g_wait(NCHUNK_T - 1, buf1, sem1)
    scat(NCHUNK_T - 1, buf1)

    plsc.subcore_barrier()
    pltpu.sync_copy(
        agg_sh.at[pl.ds(s * rpt, rpt)],
        out_hbm.at[pl.ds(c * NHALF + s * rpt, rpt)])


def _mlp_body(h_ref, agg_ref, w1_ref, b1_ref, w2_ref, b2_ref, out_ref):
    i = pl.program_id(0)
    z = h_ref[...] + agg_ref[...]
    a = jnp.maximum(
        jnp.dot(z, w1_ref[...], preferred_element_type=jnp.float32)
        + b1_ref[...], 0.0)
    y = jnp.maximum(
        jnp.dot(a, w2_ref[...], preferred_element_type=jnp.float32)
        + b2_ref[...], 0.0)
    rows = lax.broadcasted_iota(jnp.int32, (BLK_M, D), 0) + i * BLK_M
    out_ref[...] = jnp.where(rows < N, y, 0.0)


def _mlp(h, agg, W1, b1, W2, b2):
    return pl.pallas_call(
        _mlp_body,
        grid=(NBLK_M,),
        in_specs=[
            pl.BlockSpec((BLK_M, D), lambda i: (i, 0)),
            pl.BlockSpec((BLK_M, D), lambda i: (i, 0)),
            pl.BlockSpec((D, D), lambda i: (0, 0)),
            pl.BlockSpec((1, D), lambda i: (0, 0)),
            pl.BlockSpec((D, D), lambda i: (0, 0)),
            pl.BlockSpec((1, D), lambda i: (0, 0)),
        ],
        out_specs=pl.BlockSpec((BLK_M, D), lambda i: (i, 0)),
        out_shape=jax.ShapeDtypeStruct((NPAD, D), jnp.float32),
    )(h, agg, W1, b1.reshape(1, D), W2, b2.reshape(1, D))


def _readout_body(h_ref, bidx_ref, wl_ref, bl_ref, out_ref, sums_ref, cnt_ref):
    i = pl.program_id(0)

    @pl.when(i == 0)
    def _init():
        sums_ref[...] = jnp.zeros_like(sums_ref)
        cnt_ref[...] = jnp.zeros_like(cnt_ref)

    y = h_ref[...]
    bidx = bidx_ref[0]
    gid = lax.broadcasted_iota(jnp.int32, (G, BLK), 0)
    m = (gid == bidx).astype(jnp.float32)
    sums_ref[...] += jnp.dot(m, y, preferred_element_type=jnp.float32,
                             precision=lax.Precision.HIGHEST)
    cnt_ref[...] += jnp.broadcast_to(
        jnp.sum(m, axis=1, keepdims=True), (G, D))

    @pl.when(i == NBLK - 1)
    def _fin():
        mean = sums_ref[...] / jnp.maximum(cnt_ref[...], 1.0)
        proj = jnp.dot(mean, wl_ref[...], preferred_element_type=jnp.float32)
        out_ref[...] = jax.nn.sigmoid(proj + bl_ref[...])


def _readout(h, bidx3d, wl_pad, bl11):
    return pl.pallas_call(
        _readout_body,
        grid=(NBLK,),
        in_specs=[
            pl.BlockSpec((BLK, D), lambda i: (i, 0)),
            pl.BlockSpec((1, 1, BLK), lambda i: (i, 0, 0)),
            pl.BlockSpec((D, D), lambda i: (0, 0)),
            pl.BlockSpec((1, 1), lambda i: (0, 0)),
        ],
        out_specs=pl.BlockSpec((G, D), lambda i: (0, 0)),
        out_shape=jax.ShapeDtypeStruct((G, D), jnp.float32),
        scratch_shapes=[
            pltpu.VMEM((G, D), jnp.float32),
            pltpu.VMEM((G, D), jnp.float32),
        ],
    )(h, bidx3d, wl_pad, bl11)


def kernel(x, edge_index, batch_idx, atom_emb,
           W1_1, b1_1, W2_1, b2_1, W1_2, b1_2, W2_2, b2_2,
           W1_3, b1_3, W2_3, b2_3, Wl, bl):
    offs = (jnp.arange(F, dtype=jnp.int32) * V)[None, :]
    xf = (x + offs).T
    pad = (jnp.arange(NPAD - N, dtype=jnp.int32) % (F * V))[None, :]
    xf = jnp.concatenate(
        [xf, jnp.broadcast_to(pad, (F, NPAD - N))], axis=1)
    idx_a = xf.reshape(F, NW, 4, 80).transpose(1, 0, 2, 3).reshape(NW, F * 4, 80)
    idx_fill = jnp.broadcast_to(
        (jnp.arange(80, dtype=jnp.int32) % (F * V))[None, None, :], (NW, 4, 80))
    idx_a = jnp.concatenate([idx_a, idx_fill], axis=1).reshape(NW * 40, 80)
    tab = atom_emb.reshape(F * V, D)
    h0 = _atom_encode(idx_a, tab)

    npe = E_PAD - E
    pad_src = jnp.arange(npe, dtype=jnp.int32) % N
    src_i = jnp.concatenate([edge_index[0], pad_src]).reshape(NS * NCHUNK_T, CHUNK)
    dst_i = jnp.concatenate(
        [edge_index[1], jnp.full((npe,), NPAD, jnp.int32)]
    ).reshape(NS * NCHUNK_T, CHUNK)

    bidx3d = batch_idx.reshape(NBLK, 1, BLK)
    wl_pad = jnp.pad(Wl, ((0, 0), (0, D - 1)))
    bl11 = bl.reshape(1, 1)

    ws = (jnp.stack([W1_1, W1_2, W1_3]), jnp.stack([b1_1, b1_2, b1_3]),
          jnp.stack([W2_1, W2_2, W2_3]), jnp.stack([b2_1, b2_2, b2_3]))

    def layer(h, wts):
        w1, bb1, w2, bb2 = wts
        agg = _edge_agg(h, src_i, dst_i)
        return _mlp(h, agg, w1, bb1, w2, bb2), None

    h, _ = lax.scan(layer, h0, ws)
    out_full = _readout(h, bidx3d, wl_pad, bl11)
    return out_full[:, :1]
